# baseline (speedup 1.0000x reference)
import functools

import jax
import jax.numpy as jnp
from jax import lax
from jax.experimental import pallas as pl
from jax.experimental.pallas import tpu as pltpu
from jax.experimental.pallas import tpu_sc as plsc

N = 100000
E = 6400000
NC = 2
NS = 16
NW = NC * NS
ROWS = 128

NPAD = 100352
NSL = NPAD // NS
NSL4 = NSL // 4
NSL2 = NSL // 2

WE = 2048
NRE = WE // ROWS
TW_E = E // WE
PW_E = 98

WO = 1024
TW_O = (E // 2) // WO
PW_O = 98
OPAD = NW * PW_O * WO

_mesh = plsc.VectorSubcoreMesh(core_axis_name="c", subcore_axis_name="s")


def _rsqrt16(d):
    i = lax.bitcast_convert_type(d, jnp.int32)
    i = jnp.full((16,), 0x5F3759DF, jnp.int32) - (i >> 1)
    y = lax.bitcast_convert_type(i, jnp.float32)
    half = jnp.full((16,), 0.5, jnp.float32)
    three_half = jnp.full((16,), 1.5, jnp.float32)
    hd = half * d
    y = y * (three_half - hd * y * y)
    y = y * (three_half - hd * y * y)
    y = y * (three_half - hd * y * y)
    return y


def _load2d(ef, rbase, buf, sem, nr=NRE):
    for r in range(nr):
        pltpu.async_copy(ef.at[rbase + r], buf.at[r], sem)


def _wait2d(ef, rbase, buf, sem, nr=NRE):
    for r in range(nr):
        pltpu.make_async_copy(ef.at[rbase + r], buf.at[r], sem).wait()


def _load1d(ef, rbase, buf, sem, nr=NRE):
    for r in range(nr):
        pltpu.async_copy(ef.at[rbase + r], buf.at[pl.ds(r * ROWS, ROWS)], sem)


def _wait1d(ef, rbase, buf, sem, nr=NRE):
    for r in range(nr):
        pltpu.make_async_copy(
            ef.at[rbase + r], buf.at[pl.ds(r * ROWS, ROWS)], sem).wait()


@functools.partial(
    pl.kernel,
    out_type=jax.ShapeDtypeStruct((2 * NPAD,), jnp.float32),
    mesh=_mesh,
    compiler_params=pltpu.CompilerParams(needs_layout_passes=False),
    scratch_types=[
        pltpu.VMEM_SHARED((NPAD,), jnp.float32),
        pltpu.VMEM((NRE, ROWS), jnp.int32),
        pltpu.VMEM((NRE, ROWS), jnp.int32),
        pltpu.VMEM((WE,), jnp.float32),
        pltpu.SemaphoreType.DMA,
        pltpu.SemaphoreType.DMA,
        pltpu.SemaphoreType.DMA,
    ],
)
def _k1(ef, zeros, ones, degp, acc, idxa, idxb, onesb, sla, slb, ssc):
    c = lax.axis_index("c")
    s = lax.axis_index("s")
    wid = s * NC + c
    nb = s * NSL
    pltpu.sync_copy(zeros, acc.at[pl.ds(nb, NSL)])
    pltpu.sync_copy(ones, onesb)
    plsc.subcore_barrier()

    start = PW_E * wid

    def doff(j):
        return jnp.where(j < TW_E, E // ROWS + NRE * j, 0)

    def scatter(j, buf):
        @pl.when(j < TW_E)
        def _():
            hs = [
                pltpu.async_copy(onesb.at[pl.ds(r * ROWS, ROWS)],
                                 acc.at[buf.at[r]], ssc, add=True)
                for r in range(NRE)
            ]
            for h in hs:
                h.wait()

    _load2d(ef, doff(start), idxa, sla)

    def body(i2, carry):
        ja = start + 2 * i2
        _wait2d(ef, doff(ja), idxa, sla)
        _load2d(ef, doff(ja + 1), idxb, slb)
        scatter(ja, idxa)
        _wait2d(ef, doff(ja + 1), idxb, slb)

        @pl.when(i2 < PW_E // 2 - 1)
        def _():
            _load2d(ef, doff(ja + 2), idxa, sla)

        scatter(ja + 1, idxb)
        return carry

    lax.fori_loop(0, PW_E // 2, body, 0)
    plsc.subcore_barrier()
    pltpu.sync_copy(acc.at[pl.ds(nb, NSL)], degp.at[pl.ds(c * NPAD + nb, NSL)])


@functools.partial(
    pl.kernel,
    out_type=(
        jax.ShapeDtypeStruct((2 * NPAD,), jnp.float32),
        jax.ShapeDtypeStruct((NPAD,), jnp.float32),
        jax.ShapeDtypeStruct((2 * NPAD,), jnp.float32),
    ),
    mesh=_mesh,
    compiler_params=pltpu.CompilerParams(needs_layout_passes=False),
    scratch_types=[
        pltpu.VMEM_SHARED((NPAD,), jnp.float32),
        pltpu.VMEM((NPAD,), jnp.float32),
        pltpu.VMEM((NSL4,), jnp.float32),
        pltpu.VMEM((NSL4,), jnp.float32),
        pltpu.VMEM((NSL4,), jnp.float32),
        pltpu.VMEM((NSL4,), jnp.float32),
        pltpu.VMEM((WE,), jnp.int32),
        pltpu.VMEM((WE,), jnp.int32),
        pltpu.VMEM((NRE, ROWS), jnp.int32),
        pltpu.VMEM((NRE, ROWS), jnp.int32),
        pltpu.VMEM((WE,), jnp.float32),
        pltpu.VMEM((WE,), jnp.float32),
        pltpu.SemaphoreType.DMA,
        pltpu.SemaphoreType.DMA,
        pltpu.SemaphoreType.DMA,
    ],
)
def _k2(ef, xpad, degp, zeros, ap, dinv, ucopy, a_sh, uloc,
        d0b, d1b, xb, dvb, sia, sib, dia, dib, va, vb, sla, slb, ssc):
    c = lax.axis_index("c")
    s = lax.axis_index("s")
    wid = s * NC + c
    nb = s * NSL

    pltpu.sync_copy(zeros, a_sh.at[pl.ds(nb, NSL)])

    one = jnp.full((16,), 1.0, jnp.float32)

    for h in range(4):
        hb = nb + h * NSL4
        pltpu.sync_copy(degp.at[pl.ds(hb, NSL4)], d0b)
        pltpu.sync_copy(degp.at[pl.ds(NPAD + hb, NSL4)], d1b)
        pltpu.sync_copy(xpad.at[pl.ds(hb, NSL4)], xb)

        def dense(i, carry):
            o = pl.ds(i * 16, 16)
            dv = _rsqrt16(d0b[o] + d1b[o] + one)
            dvb[o] = dv
            uloc[pl.ds(h * NSL4 + i * 16, 16)] = xb[o] * dv
            return carry

        lax.fori_loop(0, NSL4 // 16, dense, 0)

        @pl.when(c == 0)
        def _():
            pltpu.sync_copy(dvb, dinv.at[pl.ds(hb, NSL4)])

    pltpu.sync_copy(uloc.at[pl.ds(0, NSL)], ucopy.at[pl.ds(c * NPAD + nb, NSL)])
    plsc.subcore_barrier()
    pltpu.sync_copy(ucopy.at[pl.ds(c * NPAD, NPAD)], uloc)

    start = PW_E * wid

    def offs(j):
        jc = jnp.where(j < TW_E, j, 0)
        return NRE * jc, E // ROWS + NRE * jc

    def loads(j, sbuf, dbuf, sem):
        so, do = offs(j)
        _load1d(ef, so, sbuf, sem)
        _load2d(ef, do, dbuf, sem)

    def waitloads(j, sbuf, dbuf, sem):
        so, do = offs(j)
        _wait1d(ef, so, sbuf, sem)
        _wait2d(ef, do, dbuf, sem)

    def gather(sbuf, vbuf):
        def chunk(k, carry):
            o = pl.ds(k * 16, 16)
            vbuf[o] = plsc.load_gather(uloc, [sbuf[o]])
            return carry

        lax.fori_loop(0, WE // 16, chunk, 0)

    def scatter(j, vbuf, dbuf):
        @pl.when(j < TW_E)
        def _():
            hs = [
                pltpu.async_copy(vbuf.at[pl.ds(r * ROWS, ROWS)],
                                 a_sh.at[dbuf.at[r]], ssc, add=True)
                for r in range(NRE)
            ]
            for h in hs:
                h.wait()

    loads(start, sia, dia, sla)

    def body(i2, carry):
        ja = start + 2 * i2
        waitloads(ja, sia, dia, sla)
        loads(ja + 1, sib, dib, slb)
        gather(sia, va)
        scatter(ja, va, dia)
        waitloads(ja + 1, sib, dib, slb)

        @pl.when(i2 < PW_E // 2 - 1)
        def _():
            loads(ja + 2, sia, dia, sla)

        gather(sib, vb)
        scatter(ja + 1, vb, dib)
        return carry

    lax.fori_loop(0, PW_E // 2, body, 0)
    plsc.subcore_barrier()
    pltpu.sync_copy(a_sh.at[pl.ds(nb, NSL)], ap.at[pl.ds(c * NPAD + nb, NSL)])


@functools.partial(
    pl.kernel,
    out_type=(
        jax.ShapeDtypeStruct((OPAD,), jnp.float32),
        jax.ShapeDtypeStruct((2 * NPAD,), jnp.float32),
    ),
    mesh=_mesh,
    compiler_params=pltpu.CompilerParams(needs_layout_passes=False),
    scratch_types=[
        pltpu.VMEM((NPAD,), jnp.float32),
        pltpu.VMEM((NSL2,), jnp.float32),
        pltpu.VMEM((NSL2,), jnp.float32),
        pltpu.VMEM((NSL2,), jnp.float32),
        pltpu.VMEM((NSL2,), jnp.float32),
        pltpu.VMEM((64,), jnp.float32),
        pltpu.VMEM((WO,), jnp.int32),
        pltpu.VMEM((WO,), jnp.int32),
        pltpu.VMEM((WO,), jnp.int32),
        pltpu.VMEM((WO,), jnp.int32),
        pltpu.VMEM((WO,), jnp.int32),
        pltpu.VMEM((WO,), jnp.int32),
        pltpu.VMEM((WO,), jnp.int32),
        pltpu.VMEM((WO,), jnp.int32),
        pltpu.VMEM((WO,), jnp.float32),
        pltpu.VMEM((WO,), jnp.float32),
        pltpu.SemaphoreType.DMA,
        pltpu.SemaphoreType.DMA,
        pltpu.SemaphoreType.DMA,
    ],
)
def _k3(ef, dinv, ucopy, ap, pvec, out, tcopy, tloc, dvb, ub, a0b, a1b, pb,
        i1a, i2a, i3a, i4a, i1b, i2b, i3b, i4b, oba, obb, sla, slb, so):
    c = lax.axis_index("c")
    s = lax.axis_index("s")
    wid = s * NC + c
    nb = s * NSL

    pltpu.sync_copy(pvec, pb)
    w00 = pb[pl.ds(0, 16)]
    w01 = pb[pl.ds(16, 16)]
    b0 = pb[pl.ds(32, 16)]
    b1 = pb[pl.ds(48, 16)]
    zero = jnp.full((16,), 0.0, jnp.float32)

    for h in range(2):
        hb = nb + h * NSL2
        pltpu.sync_copy(dinv.at[pl.ds(hb, NSL2)], dvb)
        pltpu.sync_copy(ucopy.at[pl.ds(c * NPAD + hb, NSL2)], ub)
        pltpu.sync_copy(ap.at[pl.ds(hb, NSL2)], a0b)
        pltpu.sync_copy(ap.at[pl.ds(NPAD + hb, NSL2)], a1b)

        def dense(i, carry):
            o = pl.ds(i * 16, 16)
            g = dvb[o] * (a0b[o] + a1b[o] + ub[o])
            t = (jnp.maximum(g * w00 + b0, zero)
                 + jnp.maximum(g * w01 + b1, zero))
            tloc[pl.ds(h * NSL2 + i * 16, 16)] = t
            return carry

        lax.fori_loop(0, NSL2 // 16, dense, 0)

    pltpu.sync_copy(tloc.at[pl.ds(0, NSL)], tcopy.at[pl.ds(c * NPAD + nb, NSL)])
    plsc.subcore_barrier()
    pltpu.sync_copy(tcopy.at[pl.ds(c * NPAD, NPAD)], tloc)

    start = PW_O * wid
    nro = WO // ROWS
    r_d1 = E // ROWS
    r_s2 = (E // 2) // ROWS
    r_d2 = r_d1 + r_s2
    half = jnp.full((16,), 0.5, jnp.float32)
    one = jnp.full((16,), 1.0, jnp.float32)

    def offs(j):
        jc = jnp.where(j < TW_O, j, 0)
        return nro * jc

    def loads(j, bufs, sem):
        o = offs(j)
        for base, buf in zip((0, r_d1, r_s2, r_d2), bufs):
            _load1d(ef, base + o, buf, sem, nr=nro)

    def waitloads(j, bufs, sem):
        o = offs(j)
        for base, buf in zip((0, r_d1, r_s2, r_d2), bufs):
            _wait1d(ef, base + o, buf, sem, nr=nro)

    def compute(bufs, ob):
        b1_, b2_, b3_, b4_ = bufs

        def chunk(k, carry):
            o = pl.ds(k * 16, 16)
            z = (plsc.load_gather(tloc, [b1_[o]])
                 + plsc.load_gather(tloc, [b2_[o]])
                 + plsc.load_gather(tloc, [b3_[o]])
                 + plsc.load_gather(tloc, [b4_[o]]))
            z = half * z
            ob[o] = one / (one + jnp.exp(-z))
            return carry

        lax.fori_loop(0, WO // 16, chunk, 0)

    bufsa = (i1a, i2a, i3a, i4a)
    bufsb = (i1b, i2b, i3b, i4b)
    loads(start, bufsa, sla)

    def body(i2, carry):
        ja = start + 2 * i2
        waitloads(ja, bufsa, sla)
        loads(ja + 1, bufsb, slb)
        compute(bufsa, oba)
        ha = pltpu.async_copy(oba, out.at[pl.ds(WO * ja, WO)], so)
        waitloads(ja + 1, bufsb, slb)

        @pl.when(i2 < PW_O // 2 - 1)
        def _():
            loads(ja + 2, bufsa, sla)

        compute(bufsb, obb)
        hb = pltpu.async_copy(obb, out.at[pl.ds(WO * (ja + 1), WO)], so)
        ha.wait()
        hb.wait()
        return carry

    lax.fori_loop(0, PW_O // 2, body, 0)


def kernel(x, edge_index, W, b):
    ef = edge_index.reshape((2 * E) // ROWS, ROWS)
    xpad = jnp.concatenate(
        [x.reshape(N), jnp.zeros((NPAD - N,), jnp.float32)])
    zeros = jnp.zeros((NSL,), jnp.float32)
    ones = jnp.ones((WE,), jnp.float32)
    pvec = jnp.concatenate([
        jnp.full((16,), W[0, 0], jnp.float32),
        jnp.full((16,), W[0, 1], jnp.float32),
        jnp.full((16,), b[0], jnp.float32),
        jnp.full((16,), b[1], jnp.float32),
    ])

    degp = _k1(ef, zeros, ones)
    ap, dinv, ucopy = _k2(ef, xpad, degp, zeros)
    scores, _ = _k3(ef, dinv, ucopy, ap, pvec)
    return scores[:E // 2][:, None]


# speedup vs baseline: 1.2251x; 1.2251x over previous
import functools

import jax
import jax.numpy as jnp
from jax import lax
from jax.experimental import pallas as pl
from jax.experimental.pallas import tpu as pltpu
from jax.experimental.pallas import tpu_sc as plsc

N = 100000
E = 6400000
NC = 2
NS = 16
NW = NC * NS
ROWS = 128

NPAD = 100352
NSL = NPAD // NS
NSL4 = NSL // 4
NSL2 = NSL // 2

WE = 2048
NRE = WE // ROWS
TW_E = E // WE
PW_E = 98

WO = 1024
TW_O = (E // 2) // WO
PW_O = 98
OPAD = NW * PW_O * WO

_mesh = plsc.VectorSubcoreMesh(core_axis_name="c", subcore_axis_name="s")


def _rsqrt16(d):
    i = lax.bitcast_convert_type(d, jnp.int32)
    i = jnp.full((16,), 0x5F3759DF, jnp.int32) - (i >> 1)
    y = lax.bitcast_convert_type(i, jnp.float32)
    half = jnp.full((16,), 0.5, jnp.float32)
    three_half = jnp.full((16,), 1.5, jnp.float32)
    hd = half * d
    y = y * (three_half - hd * y * y)
    y = y * (three_half - hd * y * y)
    y = y * (three_half - hd * y * y)
    return y


def _load2d(ef1, base, buf, sem):
    for r in range(NRE):
        pltpu.async_copy(ef1.at[pl.ds(base + r * ROWS, ROWS)], buf.at[r], sem)


def _wait2d(ef1, base, buf, sem):
    for r in range(NRE):
        pltpu.make_async_copy(
            ef1.at[pl.ds(base + r * ROWS, ROWS)], buf.at[r], sem).wait()


@functools.partial(
    pl.kernel,
    out_type=jax.ShapeDtypeStruct((2 * NPAD,), jnp.float32),
    mesh=_mesh,
    compiler_params=pltpu.CompilerParams(needs_layout_passes=False),
    scratch_types=[
        pltpu.VMEM_SHARED((NPAD,), jnp.float32),
        pltpu.VMEM((NRE, ROWS), jnp.int32),
        pltpu.VMEM((NRE, ROWS), jnp.int32),
        pltpu.VMEM((WE,), jnp.float32),
        pltpu.SemaphoreType.DMA,
        pltpu.SemaphoreType.DMA,
        pltpu.SemaphoreType.DMA,
    ],
)
def _k1(ef1, zeros, ones, degp, acc, idxa, idxb, onesb, sla, slb, ssc):
    c = lax.axis_index("c")
    s = lax.axis_index("s")
    wid = s * NC + c
    nb = s * NSL
    pltpu.sync_copy(zeros, acc.at[pl.ds(nb, NSL)])
    pltpu.sync_copy(ones, onesb)
    plsc.subcore_barrier()

    start = PW_E * wid

    def doff(j):
        return jnp.where(j < TW_E, E + WE * j, 0)

    def scatter(j, buf):
        @pl.when(j < TW_E)
        def _():
            hs = [
                pltpu.async_copy(onesb.at[pl.ds(r * ROWS, ROWS)],
                                 acc.at[buf.at[r]], ssc, add=True)
                for r in range(NRE)
            ]
            for h in hs:
                h.wait()

    _load2d(ef1, doff(start), idxa, sla)

    def body(i2, carry):
        ja = start + 2 * i2
        _wait2d(ef1, doff(ja), idxa, sla)
        _load2d(ef1, doff(ja + 1), idxb, slb)
        scatter(ja, idxa)
        _wait2d(ef1, doff(ja + 1), idxb, slb)

        @pl.when(i2 < PW_E // 2 - 1)
        def _():
            _load2d(ef1, doff(ja + 2), idxa, sla)

        scatter(ja + 1, idxb)
        return carry

    lax.fori_loop(0, PW_E // 2, body, 0)
    plsc.subcore_barrier()
    pltpu.sync_copy(acc.at[pl.ds(nb, NSL)], degp.at[pl.ds(c * NPAD + nb, NSL)])


@functools.partial(
    pl.kernel,
    out_type=(
        jax.ShapeDtypeStruct((2 * NPAD,), jnp.float32),
        jax.ShapeDtypeStruct((NPAD,), jnp.float32),
        jax.ShapeDtypeStruct((2 * NPAD,), jnp.float32),
    ),
    mesh=_mesh,
    compiler_params=pltpu.CompilerParams(needs_layout_passes=False),
    scratch_types=[
        pltpu.VMEM_SHARED((NPAD,), jnp.float32),
        pltpu.VMEM((NPAD,), jnp.float32),
        pltpu.VMEM((NSL4,), jnp.float32),
        pltpu.VMEM((NSL4,), jnp.float32),
        pltpu.VMEM((NSL4,), jnp.float32),
        pltpu.VMEM((NSL4,), jnp.float32),
        pltpu.VMEM((WE,), jnp.int32),
        pltpu.VMEM((WE,), jnp.int32),
        pltpu.VMEM((NRE, ROWS), jnp.int32),
        pltpu.VMEM((NRE, ROWS), jnp.int32),
        pltpu.VMEM((WE,), jnp.float32),
        pltpu.VMEM((WE,), jnp.float32),
        pltpu.SemaphoreType.DMA,
        pltpu.SemaphoreType.DMA,
        pltpu.SemaphoreType.DMA,
    ],
)
def _k2(ef1, xpad, degp, zeros, ap, dinv, ucopy, a_sh, uloc,
        d0b, d1b, xb, dvb, sia, sib, dia, dib, va, vb, sla, slb, ssc):
    c = lax.axis_index("c")
    s = lax.axis_index("s")
    wid = s * NC + c
    nb = s * NSL

    pltpu.sync_copy(zeros, a_sh.at[pl.ds(nb, NSL)])

    one = jnp.full((16,), 1.0, jnp.float32)

    for h in range(4):
        hb = nb + h * NSL4
        pltpu.sync_copy(degp.at[pl.ds(hb, NSL4)], d0b)
        pltpu.sync_copy(degp.at[pl.ds(NPAD + hb, NSL4)], d1b)
        pltpu.sync_copy(xpad.at[pl.ds(hb, NSL4)], xb)

        def dense(i, carry):
            o = pl.ds(i * 16, 16)
            dv = _rsqrt16(d0b[o] + d1b[o] + one)
            dvb[o] = dv
            uloc[pl.ds(h * NSL4 + i * 16, 16)] = xb[o] * dv
            return carry

        lax.fori_loop(0, NSL4 // 16, dense, 0)

        @pl.when(c == 0)
        def _():
            pltpu.sync_copy(dvb, dinv.at[pl.ds(hb, NSL4)])

    pltpu.sync_copy(uloc.at[pl.ds(0, NSL)], ucopy.at[pl.ds(c * NPAD + nb, NSL)])
    plsc.subcore_barrier()
    pltpu.sync_copy(ucopy.at[pl.ds(c * NPAD, NPAD)], uloc)

    start = PW_E * wid

    def offs(j):
        jc = jnp.where(j < TW_E, j, 0)
        return WE * jc, E + WE * jc

    def loads(j, sbuf, dbuf, sem):
        so, do = offs(j)
        pltpu.async_copy(ef1.at[pl.ds(so, WE)], sbuf, sem)
        _load2d(ef1, do, dbuf, sem)

    def waitloads(j, sbuf, dbuf, sem):
        so, do = offs(j)
        pltpu.make_async_copy(ef1.at[pl.ds(so, WE)], sbuf, sem).wait()
        _wait2d(ef1, do, dbuf, sem)

    def gather(sbuf, vbuf):
        def chunk(k, carry):
            for u in range(2):
                o = pl.ds(k * 32 + u * 16, 16)
                vbuf[o] = plsc.load_gather(uloc, [sbuf[o]])
            return carry

        lax.fori_loop(0, WE // 32, chunk, 0)

    def scatter(j, vbuf, dbuf):
        @pl.when(j < TW_E)
        def _():
            hs = [
                pltpu.async_copy(vbuf.at[pl.ds(r * ROWS, ROWS)],
                                 a_sh.at[dbuf.at[r]], ssc, add=True)
                for r in range(NRE)
            ]
            for h in hs:
                h.wait()

    loads(start, sia, dia, sla)

    def body(i2, carry):
        ja = start + 2 * i2
        waitloads(ja, sia, dia, sla)
        loads(ja + 1, sib, dib, slb)
        gather(sia, va)
        scatter(ja, va, dia)
        waitloads(ja + 1, sib, dib, slb)

        @pl.when(i2 < PW_E // 2 - 1)
        def _():
            loads(ja + 2, sia, dia, sla)

        gather(sib, vb)
        scatter(ja + 1, vb, dib)
        return carry

    lax.fori_loop(0, PW_E // 2, body, 0)
    plsc.subcore_barrier()
    pltpu.sync_copy(a_sh.at[pl.ds(nb, NSL)], ap.at[pl.ds(c * NPAD + nb, NSL)])


@functools.partial(
    pl.kernel,
    out_type=(
        jax.ShapeDtypeStruct((OPAD,), jnp.float32),
        jax.ShapeDtypeStruct((2 * NPAD,), jnp.float32),
    ),
    mesh=_mesh,
    compiler_params=pltpu.CompilerParams(needs_layout_passes=False),
    scratch_types=[
        pltpu.VMEM((NPAD,), jnp.float32),
        pltpu.VMEM((NSL2,), jnp.float32),
        pltpu.VMEM((NSL2,), jnp.float32),
        pltpu.VMEM((NSL2,), jnp.float32),
        pltpu.VMEM((NSL2,), jnp.float32),
        pltpu.VMEM((64,), jnp.float32),
        pltpu.VMEM((WO,), jnp.int32),
        pltpu.VMEM((WO,), jnp.int32),
        pltpu.VMEM((WO,), jnp.int32),
        pltpu.VMEM((WO,), jnp.int32),
        pltpu.VMEM((WO,), jnp.int32),
        pltpu.VMEM((WO,), jnp.int32),
        pltpu.VMEM((WO,), jnp.int32),
        pltpu.VMEM((WO,), jnp.int32),
        pltpu.VMEM((WO,), jnp.float32),
        pltpu.VMEM((WO,), jnp.float32),
        pltpu.SemaphoreType.DMA,
        pltpu.SemaphoreType.DMA,
        pltpu.SemaphoreType.DMA,
    ],
)
def _k3(ef1, dinv, ucopy, ap, pvec, out, tcopy, tloc, dvb, ub, a0b, a1b, pb,
        i1a, i2a, i3a, i4a, i1b, i2b, i3b, i4b, oba, obb, sla, slb, so):
    c = lax.axis_index("c")
    s = lax.axis_index("s")
    wid = s * NC + c
    nb = s * NSL

    pltpu.sync_copy(pvec, pb)
    w00 = pb[pl.ds(0, 16)]
    w01 = pb[pl.ds(16, 16)]
    b0 = pb[pl.ds(32, 16)]
    b1 = pb[pl.ds(48, 16)]
    zero = jnp.full((16,), 0.0, jnp.float32)

    for h in range(2):
        hb = nb + h * NSL2
        pltpu.sync_copy(dinv.at[pl.ds(hb, NSL2)], dvb)
        pltpu.sync_copy(ucopy.at[pl.ds(c * NPAD + hb, NSL2)], ub)
        pltpu.sync_copy(ap.at[pl.ds(hb, NSL2)], a0b)
        pltpu.sync_copy(ap.at[pl.ds(NPAD + hb, NSL2)], a1b)

        def dense(i, carry):
            o = pl.ds(i * 16, 16)
            g = dvb[o] * (a0b[o] + a1b[o] + ub[o])
            t = (jnp.maximum(g * w00 + b0, zero)
                 + jnp.maximum(g * w01 + b1, zero))
            tloc[pl.ds(h * NSL2 + i * 16, 16)] = t
            return carry

        lax.fori_loop(0, NSL2 // 16, dense, 0)

    pltpu.sync_copy(tloc.at[pl.ds(0, NSL)], tcopy.at[pl.ds(c * NPAD + nb, NSL)])
    plsc.subcore_barrier()
    pltpu.sync_copy(tcopy.at[pl.ds(c * NPAD, NPAD)], tloc)

    start = PW_O * wid
    o_d1 = E
    o_s2 = E // 2
    o_d2 = E + E // 2
    half = jnp.full((16,), 0.5, jnp.float32)
    one = jnp.full((16,), 1.0, jnp.float32)

    def offs(j):
        jc = jnp.where(j < TW_O, j, 0)
        return WO * jc

    def loads(j, bufs, sem):
        o = offs(j)
        for base, buf in zip((0, o_d1, o_s2, o_d2), bufs):
            pltpu.async_copy(ef1.at[pl.ds(base + o, WO)], buf, sem)

    def waitloads(j, bufs, sem):
        o = offs(j)
        for base, buf in zip((0, o_d1, o_s2, o_d2), bufs):
            pltpu.make_async_copy(ef1.at[pl.ds(base + o, WO)], buf, sem).wait()

    def compute(bufs, ob):
        b1_, b2_, b3_, b4_ = bufs

        def chunk(k, carry):
            for u in range(2):
                o = pl.ds(k * 32 + u * 16, 16)
                z = (plsc.load_gather(tloc, [b1_[o]])
                     + plsc.load_gather(tloc, [b2_[o]])
                     + plsc.load_gather(tloc, [b3_[o]])
                     + plsc.load_gather(tloc, [b4_[o]]))
                z = half * z
                ob[o] = one / (one + jnp.exp(-z))
            return carry

        lax.fori_loop(0, WO // 32, chunk, 0)

    bufsa = (i1a, i2a, i3a, i4a)
    bufsb = (i1b, i2b, i3b, i4b)
    loads(start, bufsa, sla)

    def body(i2, carry):
        ja = start + 2 * i2
        waitloads(ja, bufsa, sla)
        loads(ja + 1, bufsb, slb)
        compute(bufsa, oba)
        ha = pltpu.async_copy(oba, out.at[pl.ds(WO * ja, WO)], so)
        waitloads(ja + 1, bufsb, slb)

        @pl.when(i2 < PW_O // 2 - 1)
        def _():
            loads(ja + 2, bufsa, sla)

        compute(bufsb, obb)
        hb = pltpu.async_copy(obb, out.at[pl.ds(WO * (ja + 1), WO)], so)
        ha.wait()
        hb.wait()
        return carry

    lax.fori_loop(0, PW_O // 2, body, 0)


def kernel(x, edge_index, W, b):
    ef1 = edge_index.reshape(2 * E)
    xpad = jnp.concatenate(
        [x.reshape(N), jnp.zeros((NPAD - N,), jnp.float32)])
    zeros = jnp.zeros((NSL,), jnp.float32)
    ones = jnp.ones((WE,), jnp.float32)
    pvec = jnp.concatenate([
        jnp.full((16,), W[0, 0], jnp.float32),
        jnp.full((16,), W[0, 1], jnp.float32),
        jnp.full((16,), b[0], jnp.float32),
        jnp.full((16,), b[1], jnp.float32),
    ])

    degp = _k1(ef1, zeros, ones)
    ap, dinv, ucopy = _k2(ef1, xpad, degp, zeros)
    scores, _ = _k3(ef1, dinv, ucopy, ap, pvec)
    return scores[:E // 2][:, None]


# speedup vs baseline: 1.2378x; 1.0104x over previous
import functools

import jax
import jax.numpy as jnp
from jax import lax
from jax.experimental import pallas as pl
from jax.experimental.pallas import tpu as pltpu
from jax.experimental.pallas import tpu_sc as plsc

N = 100000
E = 6400000
NC = 2
NS = 16
NW = NC * NS
ROWS = 128

NPAD = 100352
NSL = NPAD // NS
NSL4 = NSL // 4
NSL2 = NSL // 2

WE = 2048
NRE = WE // ROWS
TW_E = E // WE
PW_E = 98

WO = 1024
TW_O = (E // 2) // WO
PW_O = 98
OPAD = NW * PW_O * WO

_mesh = plsc.VectorSubcoreMesh(core_axis_name="c", subcore_axis_name="s")


def _rsqrt16(d):
    i = lax.bitcast_convert_type(d, jnp.int32)
    i = jnp.full((16,), 0x5F3759DF, jnp.int32) - (i >> 1)
    y = lax.bitcast_convert_type(i, jnp.float32)
    half = jnp.full((16,), 0.5, jnp.float32)
    three_half = jnp.full((16,), 1.5, jnp.float32)
    hd = half * d
    y = y * (three_half - hd * y * y)
    y = y * (three_half - hd * y * y)
    y = y * (three_half - hd * y * y)
    return y


def _load2d(ef1, base, buf, sem):
    for r in range(NRE):
        pltpu.async_copy(ef1.at[pl.ds(base + r * ROWS, ROWS)], buf.at[r], sem)


def _wait2d(ef1, base, buf, sem):
    for r in range(NRE):
        pltpu.make_async_copy(
            ef1.at[pl.ds(base + r * ROWS, ROWS)], buf.at[r], sem).wait()


@functools.partial(
    pl.kernel,
    out_type=jax.ShapeDtypeStruct((2 * NPAD,), jnp.float32),
    mesh=_mesh,
    compiler_params=pltpu.CompilerParams(needs_layout_passes=False),
    scratch_types=[
        pltpu.VMEM_SHARED((NPAD,), jnp.float32),
        pltpu.VMEM((NRE, ROWS), jnp.int32),
        pltpu.VMEM((NRE, ROWS), jnp.int32),
        pltpu.VMEM((WE,), jnp.float32),
        pltpu.SemaphoreType.DMA,
        pltpu.SemaphoreType.DMA,
        pltpu.SemaphoreType.DMA,
    ],
)
def _k1(ef1, zeros, ones, degp, acc, idxa, idxb, onesb, sla, slb, ssc):
    c = lax.axis_index("c")
    s = lax.axis_index("s")
    wid = s * NC + c
    nb = s * NSL
    pltpu.sync_copy(zeros, acc.at[pl.ds(nb, NSL)])
    pltpu.sync_copy(ones, onesb)
    plsc.subcore_barrier()

    start = PW_E * wid

    def doff(j):
        return jnp.where(j < TW_E, E + WE * j, 0)

    def scatter(j, buf):
        @pl.when(j < TW_E)
        def _():
            hs = [
                pltpu.async_copy(onesb.at[pl.ds(r * ROWS, ROWS)],
                                 acc.at[buf.at[r]], ssc, add=True)
                for r in range(NRE)
            ]
            for h in hs:
                h.wait()

    _load2d(ef1, doff(start), idxa, sla)

    def body(i2, carry):
        ja = start + 2 * i2
        _wait2d(ef1, doff(ja), idxa, sla)
        _load2d(ef1, doff(ja + 1), idxb, slb)
        scatter(ja, idxa)
        _wait2d(ef1, doff(ja + 1), idxb, slb)

        @pl.when(i2 < PW_E // 2 - 1)
        def _():
            _load2d(ef1, doff(ja + 2), idxa, sla)

        scatter(ja + 1, idxb)
        return carry

    lax.fori_loop(0, PW_E // 2, body, 0)
    plsc.subcore_barrier()
    pltpu.sync_copy(acc.at[pl.ds(nb, NSL)], degp.at[pl.ds(c * NPAD + nb, NSL)])


@functools.partial(
    pl.kernel,
    out_type=(
        jax.ShapeDtypeStruct((2 * NPAD,), jnp.float32),
        jax.ShapeDtypeStruct((NPAD,), jnp.float32),
        jax.ShapeDtypeStruct((2 * NPAD,), jnp.float32),
    ),
    mesh=_mesh,
    compiler_params=pltpu.CompilerParams(needs_layout_passes=False),
    scratch_types=[
        pltpu.VMEM_SHARED((NPAD,), jnp.float32),
        pltpu.VMEM((NPAD,), jnp.float32),
        pltpu.VMEM((NSL4,), jnp.float32),
        pltpu.VMEM((NSL4,), jnp.float32),
        pltpu.VMEM((NSL4,), jnp.float32),
        pltpu.VMEM((NSL4,), jnp.float32),
        pltpu.VMEM((WE,), jnp.int32),
        pltpu.VMEM((WE,), jnp.int32),
        pltpu.VMEM((NRE, ROWS), jnp.int32),
        pltpu.VMEM((NRE, ROWS), jnp.int32),
        pltpu.VMEM((WE,), jnp.float32),
        pltpu.VMEM((WE,), jnp.float32),
        pltpu.SemaphoreType.DMA,
        pltpu.SemaphoreType.DMA,
        pltpu.SemaphoreType.DMA,
    ],
)
def _k2(ef1, xpad, degp, zeros, ap, dinv, ucopy, a_sh, uloc,
        d0b, d1b, xb, dvb, sia, sib, dia, dib, va, vb, sla, slb, ssc):
    c = lax.axis_index("c")
    s = lax.axis_index("s")
    wid = s * NC + c
    nb = s * NSL

    pltpu.sync_copy(zeros, a_sh.at[pl.ds(nb, NSL)])

    one = jnp.full((16,), 1.0, jnp.float32)

    for h in range(4):
        hb = nb + h * NSL4
        pltpu.sync_copy(degp.at[pl.ds(hb, NSL4)], d0b)
        pltpu.sync_copy(degp.at[pl.ds(NPAD + hb, NSL4)], d1b)
        pltpu.sync_copy(xpad.at[pl.ds(hb, NSL4)], xb)

        def dense(i, carry):
            o = pl.ds(i * 16, 16)
            dv = _rsqrt16(d0b[o] + d1b[o] + one)
            dvb[o] = dv
            uloc[pl.ds(h * NSL4 + i * 16, 16)] = xb[o] * dv
            return carry

        lax.fori_loop(0, NSL4 // 16, dense, 0)

        @pl.when(c == 0)
        def _():
            pltpu.sync_copy(dvb, dinv.at[pl.ds(hb, NSL4)])

    pltpu.sync_copy(uloc.at[pl.ds(0, NSL)], ucopy.at[pl.ds(c * NPAD + nb, NSL)])
    plsc.subcore_barrier()
    pltpu.sync_copy(ucopy.at[pl.ds(c * NPAD, NPAD)], uloc)

    start = PW_E * wid

    def offs(j):
        jc = jnp.where(j < TW_E, j, 0)
        return WE * jc, E + WE * jc

    def loads(j, sbuf, dbuf, sem):
        so, do = offs(j)
        pltpu.async_copy(ef1.at[pl.ds(so, WE)], sbuf, sem)
        _load2d(ef1, do, dbuf, sem)

    def waitloads(j, sbuf, dbuf, sem):
        so, do = offs(j)
        pltpu.make_async_copy(ef1.at[pl.ds(so, WE)], sbuf, sem).wait()
        _wait2d(ef1, do, dbuf, sem)

    def gather(sbuf, vbuf):
        def chunk(k, carry):
            for u in range(4):
                o = pl.ds(k * 64 + u * 16, 16)
                vbuf[o] = plsc.load_gather(uloc, [sbuf[o]])
            return carry

        lax.fori_loop(0, WE // 64, chunk, 0)

    def scatter(j, vbuf, dbuf):
        @pl.when(j < TW_E)
        def _():
            hs = [
                pltpu.async_copy(vbuf.at[pl.ds(r * ROWS, ROWS)],
                                 a_sh.at[dbuf.at[r]], ssc, add=True)
                for r in range(NRE)
            ]
            for h in hs:
                h.wait()

    loads(start, sia, dia, sla)

    def body(i2, carry):
        ja = start + 2 * i2
        waitloads(ja, sia, dia, sla)
        loads(ja + 1, sib, dib, slb)
        gather(sia, va)
        scatter(ja, va, dia)
        waitloads(ja + 1, sib, dib, slb)

        @pl.when(i2 < PW_E // 2 - 1)
        def _():
            loads(ja + 2, sia, dia, sla)

        gather(sib, vb)
        scatter(ja + 1, vb, dib)
        return carry

    lax.fori_loop(0, PW_E // 2, body, 0)
    plsc.subcore_barrier()
    pltpu.sync_copy(a_sh.at[pl.ds(nb, NSL)], ap.at[pl.ds(c * NPAD + nb, NSL)])


@functools.partial(
    pl.kernel,
    out_type=(
        jax.ShapeDtypeStruct((OPAD,), jnp.float32),
        jax.ShapeDtypeStruct((2 * NPAD,), jnp.float32),
    ),
    mesh=_mesh,
    compiler_params=pltpu.CompilerParams(needs_layout_passes=False),
    scratch_types=[
        pltpu.VMEM((NPAD,), jnp.float32),
        pltpu.VMEM((NSL2,), jnp.float32),
        pltpu.VMEM((NSL2,), jnp.float32),
        pltpu.VMEM((NSL2,), jnp.float32),
        pltpu.VMEM((NSL2,), jnp.float32),
        pltpu.VMEM((64,), jnp.float32),
        pltpu.VMEM((WO,), jnp.int32),
        pltpu.VMEM((WO,), jnp.int32),
        pltpu.VMEM((WO,), jnp.int32),
        pltpu.VMEM((WO,), jnp.int32),
        pltpu.VMEM((WO,), jnp.int32),
        pltpu.VMEM((WO,), jnp.int32),
        pltpu.VMEM((WO,), jnp.int32),
        pltpu.VMEM((WO,), jnp.int32),
        pltpu.VMEM((WO,), jnp.float32),
        pltpu.VMEM((WO,), jnp.float32),
        pltpu.SemaphoreType.DMA,
        pltpu.SemaphoreType.DMA,
        pltpu.SemaphoreType.DMA,
    ],
)
def _k3(ef1, dinv, ucopy, ap, pvec, out, tcopy, tloc, dvb, ub, a0b, a1b, pb,
        i1a, i2a, i3a, i4a, i1b, i2b, i3b, i4b, oba, obb, sla, slb, so):
    c = lax.axis_index("c")
    s = lax.axis_index("s")
    wid = s * NC + c
    nb = s * NSL

    pltpu.sync_copy(pvec, pb)
    w00 = pb[pl.ds(0, 16)]
    w01 = pb[pl.ds(16, 16)]
    b0 = pb[pl.ds(32, 16)]
    b1 = pb[pl.ds(48, 16)]
    zero = jnp.full((16,), 0.0, jnp.float32)

    for h in range(2):
        hb = nb + h * NSL2
        pltpu.sync_copy(dinv.at[pl.ds(hb, NSL2)], dvb)
        pltpu.sync_copy(ucopy.at[pl.ds(c * NPAD + hb, NSL2)], ub)
        pltpu.sync_copy(ap.at[pl.ds(hb, NSL2)], a0b)
        pltpu.sync_copy(ap.at[pl.ds(NPAD + hb, NSL2)], a1b)

        def dense(i, carry):
            o = pl.ds(i * 16, 16)
            g = dvb[o] * (a0b[o] + a1b[o] + ub[o])
            t = (jnp.maximum(g * w00 + b0, zero)
                 + jnp.maximum(g * w01 + b1, zero))
            tloc[pl.ds(h * NSL2 + i * 16, 16)] = t
            return carry

        lax.fori_loop(0, NSL2 // 16, dense, 0)

    pltpu.sync_copy(tloc.at[pl.ds(0, NSL)], tcopy.at[pl.ds(c * NPAD + nb, NSL)])
    plsc.subcore_barrier()
    pltpu.sync_copy(tcopy.at[pl.ds(c * NPAD, NPAD)], tloc)

    start = PW_O * wid
    o_d1 = E
    o_s2 = E // 2
    o_d2 = E + E // 2
    half = jnp.full((16,), 0.5, jnp.float32)
    one = jnp.full((16,), 1.0, jnp.float32)

    def offs(j):
        jc = jnp.where(j < TW_O, j, 0)
        return WO * jc

    def loads(j, bufs, sem):
        o = offs(j)
        for base, buf in zip((0, o_d1, o_s2, o_d2), bufs):
            pltpu.async_copy(ef1.at[pl.ds(base + o, WO)], buf, sem)

    def waitloads(j, bufs, sem):
        o = offs(j)
        for base, buf in zip((0, o_d1, o_s2, o_d2), bufs):
            pltpu.make_async_copy(ef1.at[pl.ds(base + o, WO)], buf, sem).wait()

    def compute(bufs, ob):
        b1_, b2_, b3_, b4_ = bufs

        def chunk(k, carry):
            for u in range(4):
                o = pl.ds(k * 64 + u * 16, 16)
                z = (plsc.load_gather(tloc, [b1_[o]])
                     + plsc.load_gather(tloc, [b2_[o]])
                     + plsc.load_gather(tloc, [b3_[o]])
                     + plsc.load_gather(tloc, [b4_[o]]))
                z = half * z
                ob[o] = one / (one + jnp.exp(-z))
            return carry

        lax.fori_loop(0, WO // 64, chunk, 0)

    bufsa = (i1a, i2a, i3a, i4a)
    bufsb = (i1b, i2b, i3b, i4b)
    loads(start, bufsa, sla)

    def body(i2, carry):
        ja = start + 2 * i2
        waitloads(ja, bufsa, sla)
        loads(ja + 1, bufsb, slb)
        compute(bufsa, oba)
        ha = pltpu.async_copy(oba, out.at[pl.ds(WO * ja, WO)], so)
        waitloads(ja + 1, bufsb, slb)

        @pl.when(i2 < PW_O // 2 - 1)
        def _():
            loads(ja + 2, bufsa, sla)

        compute(bufsb, obb)
        hb = pltpu.async_copy(obb, out.at[pl.ds(WO * (ja + 1), WO)], so)
        ha.wait()
        hb.wait()
        return carry

    lax.fori_loop(0, PW_O // 2, body, 0)


def kernel(x, edge_index, W, b):
    ef1 = edge_index.reshape(2 * E)
    xpad = jnp.concatenate(
        [x.reshape(N), jnp.zeros((NPAD - N,), jnp.float32)])
    zeros = jnp.zeros((NSL,), jnp.float32)
    ones = jnp.ones((WE,), jnp.float32)
    pvec = jnp.concatenate([
        jnp.full((16,), W[0, 0], jnp.float32),
        jnp.full((16,), W[0, 1], jnp.float32),
        jnp.full((16,), b[0], jnp.float32),
        jnp.full((16,), b[1], jnp.float32),
    ])

    degp = _k1(ef1, zeros, ones)
    ap, dinv, ucopy = _k2(ef1, xpad, degp, zeros)
    scores, _ = _k3(ef1, dinv, ucopy, ap, pvec)
    return scores[:E // 2][:, None]
